# baseline (speedup 1.0000x reference)
import functools

import jax
import jax.numpy as jnp
from jax import lax
from jax.experimental import pallas as pl
from jax.experimental.pallas import tpu as pltpu
from jax.experimental.pallas import tpu_sc as plsc

N = 10000
E = 320000
D = 128
H = 64
C = 16
G = 64

NC = 2
NS = 16
HH = H // NC

NPAD = 10240
RPS = NPAD // NS
K = 128
TCH = 2560
CPS = TCH // NS
EPAD = TCH * K
NBUF = 8

NB = 1000
NBLK = N // NB


_MESH = plsc.VectorSubcoreMesh(core_axis_name="c", subcore_axis_name="s")
_SC_PARAMS = pltpu.CompilerParams(use_tc_tiling_on_sc=False)


@functools.partial(
    pl.kernel,
    out_type=jax.ShapeDtypeStruct((NC, NPAD, HH), jnp.float32),
    mesh=_MESH,
    compiler_params=_SC_PARAMS,
    scratch_types=[
        pltpu.VMEM((CPS, K), jnp.int32),
        pltpu.VMEM((CPS, K), jnp.int32),
        pltpu.VMEM((NBUF, K, HH), jnp.float32),
        pltpu.VMEM_SHARED((NPAD, HH), jnp.float32),
        pltpu.VMEM_SHARED((NPAD, HH), jnp.float32),
        pltpu.SemaphoreType.DMA((NBUF,)),
        pltpu.SemaphoreType.DMA((NBUF,)),
    ],
)
def _sc_agg(y_hbm, src_hbm, dst_hbm, zeros_hbm, out_hbm,
            src_i, dst_i, rows, ysp, acc, gsem, ssem):
    core = lax.axis_index("c")
    sub = lax.axis_index("s")
    pltpu.sync_copy(src_hbm.at[pl.ds(sub * CPS, CPS)], src_i)
    pltpu.sync_copy(dst_hbm.at[pl.ds(sub * CPS, CPS)], dst_i)
    pltpu.sync_copy(y_hbm.at[core].at[pl.ds(sub * RPS, RPS)],
                    ysp.at[pl.ds(sub * RPS, RPS)])
    pltpu.sync_copy(zeros_hbm, acc.at[pl.ds(sub * RPS, RPS)])
    plsc.subcore_barrier()

    for b in range(NBUF):
        pltpu.async_copy(ysp.at[src_i.at[b]], rows.at[b], gsem.at[b])

    @pl.loop(0, CPS // NBUF - 1)
    def _(o):
        c0 = o * NBUF
        for b in range(NBUF):
            pltpu.make_async_copy(ysp.at[src_i.at[c0 + b]], rows.at[b],
                                  gsem.at[b]).wait()
            pltpu.async_copy(rows.at[b], acc.at[dst_i.at[c0 + b]],
                             ssem.at[b], add=True)
        for b in range(NBUF):
            pltpu.make_async_copy(rows.at[b], acc.at[dst_i.at[c0 + b]],
                                  ssem.at[b]).wait()
            pltpu.async_copy(ysp.at[src_i.at[c0 + b + NBUF]], rows.at[b],
                             gsem.at[b])

    cl = CPS - NBUF
    for b in range(NBUF):
        pltpu.make_async_copy(ysp.at[src_i.at[cl + b]], rows.at[b],
                              gsem.at[b]).wait()
        pltpu.async_copy(rows.at[b], acc.at[dst_i.at[cl + b]],
                         ssem.at[b], add=True)
    for b in range(NBUF):
        pltpu.make_async_copy(rows.at[b], acc.at[dst_i.at[cl + b]],
                              ssem.at[b]).wait()

    plsc.subcore_barrier()
    pltpu.sync_copy(acc.at[pl.ds(sub * RPS, RPS)],
                    out_hbm.at[core].at[pl.ds(sub * RPS, RPS)])


@functools.partial(
    pl.kernel,
    out_type=jax.ShapeDtypeStruct((NC, NPAD, 16), jnp.float32),
    mesh=_MESH,
    compiler_params=_SC_PARAMS,
    scratch_types=[
        pltpu.VMEM((CPS, K), jnp.int32),
        pltpu.VMEM((K, 16), jnp.float32),
        pltpu.VMEM_SHARED((NPAD, 16), jnp.float32),
    ],
)
def _sc_deg(dst_hbm, ones_hbm, zeros_hbm, out_hbm, dst_i, ones_v, acc):
    core = lax.axis_index("c")
    sub = lax.axis_index("s")
    pltpu.sync_copy(dst_hbm.at[pl.ds((core * NS + sub) * (CPS // NC),
                                     CPS // NC)], dst_i.at[pl.ds(0, CPS // NC)])
    pltpu.sync_copy(zeros_hbm, acc.at[pl.ds(sub * RPS, RPS)])
    pltpu.sync_copy(ones_hbm, ones_v)
    plsc.subcore_barrier()

    @pl.loop(0, CPS // NC)
    def _(i):
        pltpu.sync_copy(ones_v, acc.at[dst_i.at[i]], add=True)

    plsc.subcore_barrier()
    pltpu.sync_copy(acc.at[pl.ds(sub * RPS, RPS)],
                    out_hbm.at[core].at[pl.ds(sub * RPS, RPS)])


def _tc_mm1(x, W):
    def body(x_ref, w_ref, o_ref):
        o_ref[...] = jnp.dot(x_ref[...], w_ref[...],
                             preferred_element_type=jnp.float32)

    return pl.pallas_call(
        body,
        grid=(NBLK,),
        in_specs=[pl.BlockSpec((NB, D), lambda i: (i, 0)),
                  pl.BlockSpec((D, H), lambda i: (0, 0))],
        out_specs=pl.BlockSpec((NB, H), lambda i: (i, 0)),
        out_shape=jax.ShapeDtypeStruct((N, H), jnp.float32),
    )(x, W)


def _tc_prep(parts16, xw):
    def body(p_ref, xw_ref, dinv_ref, y_ref):
        deg = p_ref[0, :, 0] + p_ref[1, :, 0] + 1.0
        div = lax.rsqrt(deg)[:, None]
        dinv_ref[...] = div
        y = xw_ref[...] * div
        y_ref[0] = y[:, :HH]
        y_ref[1] = y[:, HH:]

    return pl.pallas_call(
        body,
        grid=(NBLK,),
        in_specs=[pl.BlockSpec((NC, NB, 16), lambda i: (0, i, 0)),
                  pl.BlockSpec((NB, H), lambda i: (i, 0))],
        out_specs=[pl.BlockSpec((NB, 1), lambda i: (i, 0)),
                   pl.BlockSpec((NC, NB, HH), lambda i: (0, i, 0))],
        out_shape=[jax.ShapeDtypeStruct((N, 1), jnp.float32),
                   jax.ShapeDtypeStruct((NC, N, HH), jnp.float32)],
    )(parts16, xw)


def _tc_combine(s, y, dinv, b, W):
    def body(s_ref, y_ref, dinv_ref, b_ref, w_ref, o_ref):
        div = dinv_ref[...]
        agg = jnp.concatenate([s_ref[0] + y_ref[0], s_ref[1] + y_ref[1]],
                              axis=1)
        h = div * agg + b_ref[...]
        h = jnp.maximum(h, 0.0)
        t = jnp.dot(h, w_ref[...], preferred_element_type=jnp.float32) * div
        o_ref[0] = t[:, :HH]
        o_ref[1] = t[:, HH:]

    return pl.pallas_call(
        body,
        grid=(NBLK,),
        in_specs=[pl.BlockSpec((NC, NB, HH), lambda i: (0, i, 0)),
                  pl.BlockSpec((NC, NB, HH), lambda i: (0, i, 0)),
                  pl.BlockSpec((NB, 1), lambda i: (i, 0)),
                  pl.BlockSpec((1, H), lambda i: (0, 0)),
                  pl.BlockSpec((H, H), lambda i: (0, 0))],
        out_specs=pl.BlockSpec((NC, NB, HH), lambda i: (0, i, 0)),
        out_shape=jax.ShapeDtypeStruct((NC, N, HH), jnp.float32),
    )(s, y, dinv, b, W)


def _tc_pool(s, y, dinv, b, batch3, Wl, bl):
    def body(s_ref, y_ref, dinv_ref, b_ref, bt_ref, wl_ref, bl_ref, o_ref,
             sums_ref, cnts_ref):
        i = pl.program_id(0)

        @pl.when(i == 0)
        def _():
            sums_ref[...] = jnp.zeros_like(sums_ref)
            cnts_ref[...] = jnp.zeros_like(cnts_ref)

        agg = jnp.concatenate([s_ref[0] + y_ref[0], s_ref[1] + y_ref[1]],
                              axis=1)
        h = dinv_ref[...] * agg + b_ref[...]
        bt = bt_ref[0, 0, :]
        oh = (bt[None, :] == lax.broadcasted_iota(jnp.int32, (G, NB), 0)
              ).astype(jnp.float32)
        sums_ref[...] += jnp.dot(oh, h, preferred_element_type=jnp.float32)
        cnts_ref[...] += jnp.sum(oh, axis=1, keepdims=True)

        @pl.when(i == NBLK - 1)
        def _():
            pooled = sums_ref[...] / jnp.maximum(cnts_ref[...], 1.0)
            o_ref[...] = jnp.dot(pooled, wl_ref[...],
                                 preferred_element_type=jnp.float32) + bl_ref[...]

    return pl.pallas_call(
        body,
        grid=(NBLK,),
        in_specs=[pl.BlockSpec((NC, NB, HH), lambda i: (0, i, 0)),
                  pl.BlockSpec((NC, NB, HH), lambda i: (0, i, 0)),
                  pl.BlockSpec((NB, 1), lambda i: (i, 0)),
                  pl.BlockSpec((1, H), lambda i: (0, 0)),
                  pl.BlockSpec((1, 1, NB), lambda i: (i, 0, 0)),
                  pl.BlockSpec((H, C), lambda i: (0, 0)),
                  pl.BlockSpec((1, C), lambda i: (0, 0))],
        out_specs=pl.BlockSpec((G, C), lambda i: (0, 0)),
        out_shape=jax.ShapeDtypeStruct((G, C), jnp.float32),
        scratch_shapes=[pltpu.VMEM((G, H), jnp.float32),
                        pltpu.VMEM((G, 1), jnp.float32)],
    )(s, y, dinv, b, batch3, Wl, bl)


def _pad_rows(y):
    return jnp.concatenate(
        [y, jnp.zeros((NC, NPAD - N, HH), jnp.float32)], axis=1)


def kernel(x, edge_index, batch, W1, b1, W2, b2, W3, b3, Wl, bl):
    pad = EPAD - E
    srcp = jnp.concatenate([edge_index[0],
                            jnp.full((pad,), N, jnp.int32)]).reshape(TCH, K)
    dstp = jnp.concatenate([edge_index[1],
                            jnp.full((pad,), N, jnp.int32)]).reshape(TCH, K)
    zeros_h = jnp.zeros((RPS, HH), jnp.float32)
    zeros16 = jnp.zeros((RPS, 16), jnp.float32)
    ones16 = jnp.ones((K, 16), jnp.float32)
    batch3 = batch.reshape(NBLK, 1, NB)

    parts16 = _sc_deg(dstp, ones16, zeros16)
    xw1 = _tc_mm1(x, W1)
    dinv, y1 = _tc_prep(parts16[:, :N], xw1)

    s1 = _sc_agg(_pad_rows(y1), srcp, dstp, zeros_h)
    y2 = _tc_combine(s1[:, :N], y1, dinv, b1.reshape(1, H), W2)
    s2 = _sc_agg(_pad_rows(y2), srcp, dstp, zeros_h)
    y3 = _tc_combine(s2[:, :N], y2, dinv, b2.reshape(1, H), W3)
    s3 = _sc_agg(_pad_rows(y3), srcp, dstp, zeros_h)
    return _tc_pool(s3[:, :N], y3, dinv, b3.reshape(1, H), batch3,
                    Wl, bl.reshape(1, C))


# speedup vs baseline: 1.0516x; 1.0516x over previous
import functools

import jax
import jax.numpy as jnp
from jax import lax
from jax.experimental import pallas as pl
from jax.experimental.pallas import tpu as pltpu
from jax.experimental.pallas import tpu_sc as plsc

N = 10000
E = 320000
D = 128
H = 64
C = 16
G = 64

NC = 2
NS = 16
HH = H // NC

NPAD = 10240
RPS = NPAD // NS
K = 128
TCH = 2560
CPS = TCH // NS
EPAD = TCH * K
NBUF = 2

NB = 1000
NBLK = N // NB


_MESH = plsc.VectorSubcoreMesh(core_axis_name="c", subcore_axis_name="s")
_SC_PARAMS = pltpu.CompilerParams(use_tc_tiling_on_sc=False)


@functools.partial(
    pl.kernel,
    out_type=jax.ShapeDtypeStruct((NC, NPAD, HH), jnp.float32),
    mesh=_MESH,
    compiler_params=_SC_PARAMS,
    scratch_types=[
        pltpu.VMEM((CPS, K), jnp.int32),
        pltpu.VMEM((CPS, K), jnp.int32),
        pltpu.VMEM((NBUF, K, HH), jnp.float32),
        pltpu.VMEM_SHARED((NPAD, HH), jnp.float32),
        pltpu.VMEM_SHARED((NPAD, HH), jnp.float32),
        pltpu.SemaphoreType.DMA((NBUF,)),
    ],
)
def _sc_agg(y_hbm, src_hbm, dst_hbm, zeros_hbm, out_hbm,
            src_i, dst_i, rows, ysp, acc, gsem):
    core = lax.axis_index("c")
    sub = lax.axis_index("s")
    pltpu.sync_copy(src_hbm.at[pl.ds(sub * CPS, CPS)], src_i)
    pltpu.sync_copy(dst_hbm.at[pl.ds(sub * CPS, CPS)], dst_i)
    pltpu.sync_copy(y_hbm.at[core].at[pl.ds(sub * RPS, RPS)],
                    ysp.at[pl.ds(sub * RPS, RPS)])
    pltpu.sync_copy(zeros_hbm, acc.at[pl.ds(sub * RPS, RPS)])
    plsc.subcore_barrier()

    for b in range(NBUF):
        pltpu.async_copy(ysp.at[src_i.at[b]], rows.at[b], gsem.at[b])

    @pl.loop(0, CPS // NBUF - 1)
    def _(o):
        c0 = o * NBUF
        for b in range(NBUF):
            pltpu.make_async_copy(ysp.at[src_i.at[c0 + b]], rows.at[b],
                                  gsem.at[b]).wait()
            pltpu.sync_copy(rows.at[b], acc.at[dst_i.at[c0 + b]], add=True)
            pltpu.async_copy(ysp.at[src_i.at[c0 + b + NBUF]], rows.at[b],
                             gsem.at[b])

    cl = CPS - NBUF
    for b in range(NBUF):
        pltpu.make_async_copy(ysp.at[src_i.at[cl + b]], rows.at[b],
                              gsem.at[b]).wait()
        pltpu.sync_copy(rows.at[b], acc.at[dst_i.at[cl + b]], add=True)

    plsc.subcore_barrier()
    pltpu.sync_copy(acc.at[pl.ds(sub * RPS, RPS)],
                    out_hbm.at[core].at[pl.ds(sub * RPS, RPS)])


@functools.partial(
    pl.kernel,
    out_type=jax.ShapeDtypeStruct((NC, NPAD, 16), jnp.float32),
    mesh=_MESH,
    compiler_params=_SC_PARAMS,
    scratch_types=[
        pltpu.VMEM((CPS, K), jnp.int32),
        pltpu.VMEM((K, 16), jnp.float32),
        pltpu.VMEM_SHARED((NPAD, 16), jnp.float32),
    ],
)
def _sc_deg(dst_hbm, ones_hbm, zeros_hbm, out_hbm, dst_i, ones_v, acc):
    core = lax.axis_index("c")
    sub = lax.axis_index("s")
    pltpu.sync_copy(dst_hbm.at[pl.ds((core * NS + sub) * (CPS // NC),
                                     CPS // NC)], dst_i.at[pl.ds(0, CPS // NC)])
    pltpu.sync_copy(zeros_hbm, acc.at[pl.ds(sub * RPS, RPS)])
    pltpu.sync_copy(ones_hbm, ones_v)
    plsc.subcore_barrier()

    @pl.loop(0, CPS // NC)
    def _(i):
        pltpu.sync_copy(ones_v, acc.at[dst_i.at[i]], add=True)

    plsc.subcore_barrier()
    pltpu.sync_copy(acc.at[pl.ds(sub * RPS, RPS)],
                    out_hbm.at[core].at[pl.ds(sub * RPS, RPS)])


def _tc_mm1(x, W):
    def body(x_ref, w_ref, o_ref):
        o_ref[...] = jnp.dot(x_ref[...], w_ref[...],
                             preferred_element_type=jnp.float32)

    return pl.pallas_call(
        body,
        grid=(NBLK,),
        in_specs=[pl.BlockSpec((NB, D), lambda i: (i, 0)),
                  pl.BlockSpec((D, H), lambda i: (0, 0))],
        out_specs=pl.BlockSpec((NB, H), lambda i: (i, 0)),
        out_shape=jax.ShapeDtypeStruct((N, H), jnp.float32),
    )(x, W)


def _tc_prep(parts16, xw):
    def body(p_ref, xw_ref, dinv_ref, y_ref):
        deg = p_ref[0, :, 0] + p_ref[1, :, 0] + 1.0
        div = lax.rsqrt(deg)[:, None]
        dinv_ref[...] = div
        y = xw_ref[...] * div
        y_ref[0] = y[:, :HH]
        y_ref[1] = y[:, HH:]

    return pl.pallas_call(
        body,
        grid=(NBLK,),
        in_specs=[pl.BlockSpec((NC, NB, 16), lambda i: (0, i, 0)),
                  pl.BlockSpec((NB, H), lambda i: (i, 0))],
        out_specs=[pl.BlockSpec((NB, 1), lambda i: (i, 0)),
                   pl.BlockSpec((NC, NB, HH), lambda i: (0, i, 0))],
        out_shape=[jax.ShapeDtypeStruct((N, 1), jnp.float32),
                   jax.ShapeDtypeStruct((NC, N, HH), jnp.float32)],
    )(parts16, xw)


def _tc_combine(s, y, dinv, b, W):
    def body(s_ref, y_ref, dinv_ref, b_ref, w_ref, o_ref):
        div = dinv_ref[...]
        agg = jnp.concatenate([s_ref[0] + y_ref[0], s_ref[1] + y_ref[1]],
                              axis=1)
        h = div * agg + b_ref[...]
        h = jnp.maximum(h, 0.0)
        t = jnp.dot(h, w_ref[...], preferred_element_type=jnp.float32) * div
        o_ref[0] = t[:, :HH]
        o_ref[1] = t[:, HH:]

    return pl.pallas_call(
        body,
        grid=(NBLK,),
        in_specs=[pl.BlockSpec((NC, NB, HH), lambda i: (0, i, 0)),
                  pl.BlockSpec((NC, NB, HH), lambda i: (0, i, 0)),
                  pl.BlockSpec((NB, 1), lambda i: (i, 0)),
                  pl.BlockSpec((1, H), lambda i: (0, 0)),
                  pl.BlockSpec((H, H), lambda i: (0, 0))],
        out_specs=pl.BlockSpec((NC, NB, HH), lambda i: (0, i, 0)),
        out_shape=jax.ShapeDtypeStruct((NC, N, HH), jnp.float32),
    )(s, y, dinv, b, W)


def _tc_pool(s, y, dinv, b, batch3, Wl, bl):
    def body(s_ref, y_ref, dinv_ref, b_ref, bt_ref, wl_ref, bl_ref, o_ref,
             sums_ref, cnts_ref):
        i = pl.program_id(0)

        @pl.when(i == 0)
        def _():
            sums_ref[...] = jnp.zeros_like(sums_ref)
            cnts_ref[...] = jnp.zeros_like(cnts_ref)

        agg = jnp.concatenate([s_ref[0] + y_ref[0], s_ref[1] + y_ref[1]],
                              axis=1)
        h = dinv_ref[...] * agg + b_ref[...]
        bt = bt_ref[0, 0, :]
        oh = (bt[None, :] == lax.broadcasted_iota(jnp.int32, (G, NB), 0)
              ).astype(jnp.float32)
        sums_ref[...] += jnp.dot(oh, h, preferred_element_type=jnp.float32)
        cnts_ref[...] += jnp.sum(oh, axis=1, keepdims=True)

        @pl.when(i == NBLK - 1)
        def _():
            pooled = sums_ref[...] / jnp.maximum(cnts_ref[...], 1.0)
            o_ref[...] = jnp.dot(pooled, wl_ref[...],
                                 preferred_element_type=jnp.float32) + bl_ref[...]

    return pl.pallas_call(
        body,
        grid=(NBLK,),
        in_specs=[pl.BlockSpec((NC, NB, HH), lambda i: (0, i, 0)),
                  pl.BlockSpec((NC, NB, HH), lambda i: (0, i, 0)),
                  pl.BlockSpec((NB, 1), lambda i: (i, 0)),
                  pl.BlockSpec((1, H), lambda i: (0, 0)),
                  pl.BlockSpec((1, 1, NB), lambda i: (i, 0, 0)),
                  pl.BlockSpec((H, C), lambda i: (0, 0)),
                  pl.BlockSpec((1, C), lambda i: (0, 0))],
        out_specs=pl.BlockSpec((G, C), lambda i: (0, 0)),
        out_shape=jax.ShapeDtypeStruct((G, C), jnp.float32),
        scratch_shapes=[pltpu.VMEM((G, H), jnp.float32),
                        pltpu.VMEM((G, 1), jnp.float32)],
    )(s, y, dinv, b, batch3, Wl, bl)


def _pad_rows(y):
    return jnp.concatenate(
        [y, jnp.zeros((NC, NPAD - N, HH), jnp.float32)], axis=1)


def kernel(x, edge_index, batch, W1, b1, W2, b2, W3, b3, Wl, bl):
    pad = EPAD - E
    srcp = jnp.concatenate([edge_index[0],
                            jnp.full((pad,), N, jnp.int32)]).reshape(TCH, K)
    dstp = jnp.concatenate([edge_index[1],
                            jnp.full((pad,), N, jnp.int32)]).reshape(TCH, K)
    zeros_h = jnp.zeros((RPS, HH), jnp.float32)
    zeros16 = jnp.zeros((RPS, 16), jnp.float32)
    ones16 = jnp.ones((K, 16), jnp.float32)
    batch3 = batch.reshape(NBLK, 1, NB)

    parts16 = _sc_deg(dstp, ones16, zeros16)
    xw1 = _tc_mm1(x, W1)
    dinv, y1 = _tc_prep(parts16[:, :N], xw1)

    s1 = _sc_agg(_pad_rows(y1), srcp, dstp, zeros_h)
    y2 = _tc_combine(s1[:, :N], y1, dinv, b1.reshape(1, H), W2)
    s2 = _sc_agg(_pad_rows(y2), srcp, dstp, zeros_h)
    y3 = _tc_combine(s2[:, :N], y2, dinv, b2.reshape(1, H), W3)
    s3 = _sc_agg(_pad_rows(y3), srcp, dstp, zeros_h)
    return _tc_pool(s3[:, :N], y3, dinv, b3.reshape(1, H), batch3,
                    Wl, bl.reshape(1, C))


# speedup vs baseline: 1.0798x; 1.0268x over previous
import functools

import jax
import jax.numpy as jnp
from jax import lax
from jax.experimental import pallas as pl
from jax.experimental.pallas import tpu as pltpu
from jax.experimental.pallas import tpu_sc as plsc

N = 10000
E = 320000
D = 128
H = 64
C = 16
G = 64

NC = 2
NS = 16
HH = H // NC

NPAD = 10240
RPS = NPAD // NS
K = 128
TCH = 2560
CPS = TCH // NS
EPAD = TCH * K
NBUF = 4

NB = 1000
NBLK = N // NB


_MESH = plsc.VectorSubcoreMesh(core_axis_name="c", subcore_axis_name="s")
_SC_PARAMS = pltpu.CompilerParams(use_tc_tiling_on_sc=False)


@functools.partial(
    pl.kernel,
    out_type=jax.ShapeDtypeStruct((NC, NPAD, HH), jnp.float32),
    mesh=_MESH,
    compiler_params=_SC_PARAMS,
    scratch_types=[
        pltpu.VMEM((CPS, K), jnp.int32),
        pltpu.VMEM((CPS, K), jnp.int32),
        pltpu.VMEM((NBUF, K, HH), jnp.float32),
        pltpu.VMEM_SHARED((NPAD, HH), jnp.float32),
        pltpu.VMEM_SHARED((NPAD, HH), jnp.float32),
        pltpu.SemaphoreType.DMA((NBUF,)),
    ],
)
def _sc_agg(y_hbm, src_hbm, dst_hbm, zeros_hbm, out_hbm,
            src_i, dst_i, rows, ysp, acc, gsem):
    core = lax.axis_index("c")
    sub = lax.axis_index("s")
    pltpu.async_copy(src_hbm.at[pl.ds(sub * CPS, CPS)], src_i, gsem.at[0])
    pltpu.async_copy(dst_hbm.at[pl.ds(sub * CPS, CPS)], dst_i, gsem.at[1])
    pltpu.async_copy(y_hbm.at[core].at[pl.ds(sub * RPS, RPS)],
                     ysp.at[pl.ds(sub * RPS, RPS)], gsem.at[2])
    pltpu.async_copy(zeros_hbm, acc.at[pl.ds(sub * RPS, RPS)], gsem.at[3])
    pltpu.make_async_copy(src_hbm.at[pl.ds(sub * CPS, CPS)], src_i,
                          gsem.at[0]).wait()
    pltpu.make_async_copy(dst_hbm.at[pl.ds(sub * CPS, CPS)], dst_i,
                          gsem.at[1]).wait()
    pltpu.make_async_copy(y_hbm.at[core].at[pl.ds(sub * RPS, RPS)],
                          ysp.at[pl.ds(sub * RPS, RPS)], gsem.at[2]).wait()
    pltpu.make_async_copy(zeros_hbm, acc.at[pl.ds(sub * RPS, RPS)],
                          gsem.at[3]).wait()
    plsc.subcore_barrier()

    for b in range(NBUF):
        pltpu.async_copy(ysp.at[src_i.at[b]], rows.at[b], gsem.at[b])

    @pl.loop(0, CPS // NBUF - 1)
    def _(o):
        c0 = o * NBUF
        for b in range(NBUF):
            pltpu.make_async_copy(ysp.at[src_i.at[c0 + b]], rows.at[b],
                                  gsem.at[b]).wait()
            pltpu.sync_copy(rows.at[b], acc.at[dst_i.at[c0 + b]], add=True)
            pltpu.async_copy(ysp.at[src_i.at[c0 + b + NBUF]], rows.at[b],
                             gsem.at[b])

    cl = CPS - NBUF
    for b in range(NBUF):
        pltpu.make_async_copy(ysp.at[src_i.at[cl + b]], rows.at[b],
                              gsem.at[b]).wait()
        pltpu.sync_copy(rows.at[b], acc.at[dst_i.at[cl + b]], add=True)

    plsc.subcore_barrier()
    pltpu.sync_copy(acc.at[pl.ds(sub * RPS, RPS)],
                    out_hbm.at[core].at[pl.ds(sub * RPS, RPS)])


@functools.partial(
    pl.kernel,
    out_type=jax.ShapeDtypeStruct((NC, NPAD, 16), jnp.float32),
    mesh=_MESH,
    compiler_params=_SC_PARAMS,
    scratch_types=[
        pltpu.VMEM((CPS, K), jnp.int32),
        pltpu.VMEM((K, 16), jnp.float32),
        pltpu.VMEM_SHARED((NPAD, 16), jnp.float32),
    ],
)
def _sc_deg(dst_hbm, ones_hbm, zeros_hbm, out_hbm, dst_i, ones_v, acc):
    core = lax.axis_index("c")
    sub = lax.axis_index("s")
    pltpu.sync_copy(dst_hbm.at[pl.ds((core * NS + sub) * (CPS // NC),
                                     CPS // NC)], dst_i.at[pl.ds(0, CPS // NC)])
    pltpu.sync_copy(zeros_hbm, acc.at[pl.ds(sub * RPS, RPS)])
    pltpu.sync_copy(ones_hbm, ones_v)
    plsc.subcore_barrier()

    @pl.loop(0, CPS // NC)
    def _(i):
        pltpu.sync_copy(ones_v, acc.at[dst_i.at[i]], add=True)

    plsc.subcore_barrier()
    pltpu.sync_copy(acc.at[pl.ds(sub * RPS, RPS)],
                    out_hbm.at[core].at[pl.ds(sub * RPS, RPS)])


def _tc_mm1(x, W):
    def body(x_ref, w_ref, o_ref):
        o_ref[...] = jnp.dot(x_ref[...], w_ref[...],
                             preferred_element_type=jnp.float32)

    return pl.pallas_call(
        body,
        grid=(NBLK,),
        in_specs=[pl.BlockSpec((NB, D), lambda i: (i, 0)),
                  pl.BlockSpec((D, H), lambda i: (0, 0))],
        out_specs=pl.BlockSpec((NB, H), lambda i: (i, 0)),
        out_shape=jax.ShapeDtypeStruct((N, H), jnp.float32),
    )(x, W)


def _tc_prep(parts16, xw):
    def body(p_ref, xw_ref, dinv_ref, y_ref):
        deg = p_ref[0, :, 0] + p_ref[1, :, 0] + 1.0
        div = lax.rsqrt(deg)[:, None]
        dinv_ref[...] = div
        y = xw_ref[...] * div
        y_ref[0] = y[:, :HH]
        y_ref[1] = y[:, HH:]

    return pl.pallas_call(
        body,
        grid=(NBLK,),
        in_specs=[pl.BlockSpec((NC, NB, 16), lambda i: (0, i, 0)),
                  pl.BlockSpec((NB, H), lambda i: (i, 0))],
        out_specs=[pl.BlockSpec((NB, 1), lambda i: (i, 0)),
                   pl.BlockSpec((NC, NB, HH), lambda i: (0, i, 0))],
        out_shape=[jax.ShapeDtypeStruct((N, 1), jnp.float32),
                   jax.ShapeDtypeStruct((NC, N, HH), jnp.float32)],
    )(parts16, xw)


def _tc_combine(s, y, dinv, b, W):
    def body(s_ref, y_ref, dinv_ref, b_ref, w_ref, o_ref):
        div = dinv_ref[...]
        agg = jnp.concatenate([s_ref[0] + y_ref[0], s_ref[1] + y_ref[1]],
                              axis=1)
        h = div * agg + b_ref[...]
        h = jnp.maximum(h, 0.0)
        t = jnp.dot(h, w_ref[...], preferred_element_type=jnp.float32) * div
        o_ref[0] = t[:, :HH]
        o_ref[1] = t[:, HH:]

    return pl.pallas_call(
        body,
        grid=(NBLK,),
        in_specs=[pl.BlockSpec((NC, NB, HH), lambda i: (0, i, 0)),
                  pl.BlockSpec((NC, NB, HH), lambda i: (0, i, 0)),
                  pl.BlockSpec((NB, 1), lambda i: (i, 0)),
                  pl.BlockSpec((1, H), lambda i: (0, 0)),
                  pl.BlockSpec((H, H), lambda i: (0, 0))],
        out_specs=pl.BlockSpec((NC, NB, HH), lambda i: (0, i, 0)),
        out_shape=jax.ShapeDtypeStruct((NC, N, HH), jnp.float32),
    )(s, y, dinv, b, W)


def _tc_pool(s, y, dinv, b, batch3, Wl, bl):
    def body(s_ref, y_ref, dinv_ref, b_ref, bt_ref, wl_ref, bl_ref, o_ref,
             sums_ref, cnts_ref):
        i = pl.program_id(0)

        @pl.when(i == 0)
        def _():
            sums_ref[...] = jnp.zeros_like(sums_ref)
            cnts_ref[...] = jnp.zeros_like(cnts_ref)

        agg = jnp.concatenate([s_ref[0] + y_ref[0], s_ref[1] + y_ref[1]],
                              axis=1)
        h = dinv_ref[...] * agg + b_ref[...]
        bt = bt_ref[0, 0, :]
        oh = (bt[None, :] == lax.broadcasted_iota(jnp.int32, (G, NB), 0)
              ).astype(jnp.float32)
        sums_ref[...] += jnp.dot(oh, h, preferred_element_type=jnp.float32)
        cnts_ref[...] += jnp.sum(oh, axis=1, keepdims=True)

        @pl.when(i == NBLK - 1)
        def _():
            pooled = sums_ref[...] / jnp.maximum(cnts_ref[...], 1.0)
            o_ref[...] = jnp.dot(pooled, wl_ref[...],
                                 preferred_element_type=jnp.float32) + bl_ref[...]

    return pl.pallas_call(
        body,
        grid=(NBLK,),
        in_specs=[pl.BlockSpec((NC, NB, HH), lambda i: (0, i, 0)),
                  pl.BlockSpec((NC, NB, HH), lambda i: (0, i, 0)),
                  pl.BlockSpec((NB, 1), lambda i: (i, 0)),
                  pl.BlockSpec((1, H), lambda i: (0, 0)),
                  pl.BlockSpec((1, 1, NB), lambda i: (i, 0, 0)),
                  pl.BlockSpec((H, C), lambda i: (0, 0)),
                  pl.BlockSpec((1, C), lambda i: (0, 0))],
        out_specs=pl.BlockSpec((G, C), lambda i: (0, 0)),
        out_shape=jax.ShapeDtypeStruct((G, C), jnp.float32),
        scratch_shapes=[pltpu.VMEM((G, H), jnp.float32),
                        pltpu.VMEM((G, 1), jnp.float32)],
    )(s, y, dinv, b, batch3, Wl, bl)


def _pad_rows(y):
    return jnp.concatenate(
        [y, jnp.zeros((NC, NPAD - N, HH), jnp.float32)], axis=1)


def kernel(x, edge_index, batch, W1, b1, W2, b2, W3, b3, Wl, bl):
    pad = EPAD - E
    srcp = jnp.concatenate([edge_index[0],
                            jnp.full((pad,), N, jnp.int32)]).reshape(TCH, K)
    dstp = jnp.concatenate([edge_index[1],
                            jnp.full((pad,), N, jnp.int32)]).reshape(TCH, K)
    zeros_h = jnp.zeros((RPS, HH), jnp.float32)
    zeros16 = jnp.zeros((RPS, 16), jnp.float32)
    ones16 = jnp.ones((K, 16), jnp.float32)
    batch3 = batch.reshape(NBLK, 1, NB)

    parts16 = _sc_deg(dstp, ones16, zeros16)
    xw1 = _tc_mm1(x, W1)
    dinv, y1 = _tc_prep(parts16[:, :N], xw1)

    s1 = _sc_agg(_pad_rows(y1), srcp, dstp, zeros_h)
    y2 = _tc_combine(s1[:, :N], y1, dinv, b1.reshape(1, H), W2)
    s2 = _sc_agg(_pad_rows(y2), srcp, dstp, zeros_h)
    y3 = _tc_combine(s2[:, :N], y2, dinv, b2.reshape(1, H), W3)
    s3 = _sc_agg(_pad_rows(y3), srcp, dstp, zeros_h)
    return _tc_pool(s3[:, :N], y3, dinv, b3.reshape(1, H), batch3,
                    Wl, bl.reshape(1, C))


# speedup vs baseline: 1.0812x; 1.0012x over previous
import functools

import jax
import jax.numpy as jnp
from jax import lax
from jax.experimental import pallas as pl
from jax.experimental.pallas import tpu as pltpu
from jax.experimental.pallas import tpu_sc as plsc

N = 10000
E = 320000
D = 128
H = 64
C = 16
G = 64

NC = 2
NS = 16
HH = H // NC

NPAD = 10240
RPS = NPAD // NS
K = 128
TCH = 2560
CPS = TCH // NS
EPAD = TCH * K
NBUF = 4

NB = 1000
NBLK = N // NB


_MESH = plsc.VectorSubcoreMesh(core_axis_name="c", subcore_axis_name="s")
_SC_PARAMS = pltpu.CompilerParams(use_tc_tiling_on_sc=False)


@functools.partial(
    pl.kernel,
    out_type=jax.ShapeDtypeStruct((NC, NPAD, HH), jnp.float32),
    mesh=_MESH,
    compiler_params=_SC_PARAMS,
    scratch_types=[
        pltpu.VMEM((CPS, K), jnp.int32),
        pltpu.VMEM((CPS, K), jnp.int32),
        pltpu.VMEM((NBUF, K, HH), jnp.float32),
        pltpu.VMEM_SHARED((NPAD, HH), jnp.float32),
        pltpu.VMEM_SHARED((NPAD, HH), jnp.float32),
        pltpu.SemaphoreType.DMA((NBUF,)),
    ],
)
def _sc_agg(y_hbm, src_hbm, dst_hbm, zeros_hbm, out_hbm,
            src_i, dst_i, rows, ysp, acc, gsem):
    core = lax.axis_index("c")
    sub = lax.axis_index("s")
    pltpu.async_copy(src_hbm.at[pl.ds(sub * CPS, CPS)], src_i, gsem.at[0])
    pltpu.async_copy(dst_hbm.at[pl.ds(sub * CPS, CPS)], dst_i, gsem.at[1])
    pltpu.async_copy(y_hbm.at[core].at[pl.ds(sub * RPS, RPS)],
                     ysp.at[pl.ds(sub * RPS, RPS)], gsem.at[2])
    pltpu.async_copy(zeros_hbm, acc.at[pl.ds(sub * RPS, RPS)], gsem.at[3])
    pltpu.make_async_copy(src_hbm.at[pl.ds(sub * CPS, CPS)], src_i,
                          gsem.at[0]).wait()
    pltpu.make_async_copy(dst_hbm.at[pl.ds(sub * CPS, CPS)], dst_i,
                          gsem.at[1]).wait()
    pltpu.make_async_copy(y_hbm.at[core].at[pl.ds(sub * RPS, RPS)],
                          ysp.at[pl.ds(sub * RPS, RPS)], gsem.at[2]).wait()
    pltpu.make_async_copy(zeros_hbm, acc.at[pl.ds(sub * RPS, RPS)],
                          gsem.at[3]).wait()
    plsc.subcore_barrier()

    for b in range(NBUF):
        pltpu.async_copy(ysp.at[src_i.at[b]], rows.at[b], gsem.at[b])

    @pl.loop(0, CPS // NBUF - 1)
    def _(o):
        c0 = o * NBUF
        for b in range(NBUF):
            pltpu.make_async_copy(ysp.at[src_i.at[c0 + b]], rows.at[b],
                                  gsem.at[b]).wait()
            pltpu.sync_copy(rows.at[b], acc.at[dst_i.at[c0 + b]], add=True)
            pltpu.async_copy(ysp.at[src_i.at[c0 + b + NBUF]], rows.at[b],
                             gsem.at[b])

    cl = CPS - NBUF
    for b in range(NBUF):
        pltpu.make_async_copy(ysp.at[src_i.at[cl + b]], rows.at[b],
                              gsem.at[b]).wait()
        pltpu.sync_copy(rows.at[b], acc.at[dst_i.at[cl + b]], add=True)

    plsc.subcore_barrier()
    pltpu.sync_copy(acc.at[pl.ds(sub * RPS, RPS)],
                    out_hbm.at[core].at[pl.ds(sub * RPS, RPS)])


@functools.partial(
    pl.kernel,
    out_type=jax.ShapeDtypeStruct((NC, NPAD, 16), jnp.float32),
    mesh=_MESH,
    compiler_params=_SC_PARAMS,
    scratch_types=[
        pltpu.VMEM((CPS, K), jnp.int32),
        pltpu.VMEM((K, 16), jnp.float32),
        pltpu.VMEM_SHARED((NPAD, 16), jnp.float32),
        pltpu.SemaphoreType.DMA((4,)),
    ],
)
def _sc_deg(dst_hbm, ones_hbm, zeros_hbm, out_hbm, dst_i, ones_v, acc, sem):
    core = lax.axis_index("c")
    sub = lax.axis_index("s")
    dsl = dst_hbm.at[pl.ds((core * NS + sub) * (CPS // NC), CPS // NC)]
    dvw = dst_i.at[pl.ds(0, CPS // NC)]
    zsl = acc.at[pl.ds(sub * RPS, RPS)]
    pltpu.async_copy(dsl, dvw, sem.at[0])
    pltpu.async_copy(zeros_hbm, zsl, sem.at[1])
    pltpu.async_copy(ones_hbm, ones_v, sem.at[2])
    pltpu.make_async_copy(dsl, dvw, sem.at[0]).wait()
    pltpu.make_async_copy(zeros_hbm, zsl, sem.at[1]).wait()
    pltpu.make_async_copy(ones_hbm, ones_v, sem.at[2]).wait()
    plsc.subcore_barrier()

    @pl.loop(0, CPS // NC)
    def _(i):
        pltpu.async_copy(ones_v, acc.at[dst_i.at[i]], sem.at[3], add=True)

    @pl.loop(0, CPS // NC)
    def _(i):
        pltpu.make_async_copy(ones_v, acc.at[dst_i.at[i]], sem.at[3]).wait()

    plsc.subcore_barrier()
    pltpu.sync_copy(acc.at[pl.ds(sub * RPS, RPS)],
                    out_hbm.at[core].at[pl.ds(sub * RPS, RPS)])


def _tc_mm1(x, W):
    def body(x_ref, w_ref, o_ref):
        o_ref[...] = jnp.dot(x_ref[...], w_ref[...],
                             preferred_element_type=jnp.float32)

    return pl.pallas_call(
        body,
        grid=(NBLK,),
        in_specs=[pl.BlockSpec((NB, D), lambda i: (i, 0)),
                  pl.BlockSpec((D, H), lambda i: (0, 0))],
        out_specs=pl.BlockSpec((NB, H), lambda i: (i, 0)),
        out_shape=jax.ShapeDtypeStruct((N, H), jnp.float32),
    )(x, W)


def _tc_prep(parts16, xw):
    def body(p_ref, xw_ref, dinv_ref, y_ref):
        deg = p_ref[0, :, 0] + p_ref[1, :, 0] + 1.0
        div = lax.rsqrt(deg)[:, None]
        dinv_ref[...] = div
        y = xw_ref[...] * div
        y_ref[0] = y[:, :HH]
        y_ref[1] = y[:, HH:]

    return pl.pallas_call(
        body,
        grid=(NBLK,),
        in_specs=[pl.BlockSpec((NC, NB, 16), lambda i: (0, i, 0)),
                  pl.BlockSpec((NB, H), lambda i: (i, 0))],
        out_specs=[pl.BlockSpec((NB, 1), lambda i: (i, 0)),
                   pl.BlockSpec((NC, NB, HH), lambda i: (0, i, 0))],
        out_shape=[jax.ShapeDtypeStruct((N, 1), jnp.float32),
                   jax.ShapeDtypeStruct((NC, N, HH), jnp.float32)],
    )(parts16, xw)


def _tc_combine(s, y, dinv, b, W):
    def body(s_ref, y_ref, dinv_ref, b_ref, w_ref, o_ref):
        div = dinv_ref[...]
        agg = jnp.concatenate([s_ref[0] + y_ref[0], s_ref[1] + y_ref[1]],
                              axis=1)
        h = div * agg + b_ref[...]
        h = jnp.maximum(h, 0.0)
        t = jnp.dot(h, w_ref[...], preferred_element_type=jnp.float32) * div
        o_ref[0] = t[:, :HH]
        o_ref[1] = t[:, HH:]

    return pl.pallas_call(
        body,
        grid=(NBLK,),
        in_specs=[pl.BlockSpec((NC, NB, HH), lambda i: (0, i, 0)),
                  pl.BlockSpec((NC, NB, HH), lambda i: (0, i, 0)),
                  pl.BlockSpec((NB, 1), lambda i: (i, 0)),
                  pl.BlockSpec((1, H), lambda i: (0, 0)),
                  pl.BlockSpec((H, H), lambda i: (0, 0))],
        out_specs=pl.BlockSpec((NC, NB, HH), lambda i: (0, i, 0)),
        out_shape=jax.ShapeDtypeStruct((NC, N, HH), jnp.float32),
    )(s, y, dinv, b, W)


def _tc_pool(s, y, dinv, b, batch3, Wl, bl):
    def body(s_ref, y_ref, dinv_ref, b_ref, bt_ref, wl_ref, bl_ref, o_ref,
             sums_ref, cnts_ref):
        i = pl.program_id(0)

        @pl.when(i == 0)
        def _():
            sums_ref[...] = jnp.zeros_like(sums_ref)
            cnts_ref[...] = jnp.zeros_like(cnts_ref)

        agg = jnp.concatenate([s_ref[0] + y_ref[0], s_ref[1] + y_ref[1]],
                              axis=1)
        h = dinv_ref[...] * agg + b_ref[...]
        bt = bt_ref[0, 0, :]
        oh = (bt[None, :] == lax.broadcasted_iota(jnp.int32, (G, NB), 0)
              ).astype(jnp.float32)
        sums_ref[...] += jnp.dot(oh, h, preferred_element_type=jnp.float32)
        cnts_ref[...] += jnp.sum(oh, axis=1, keepdims=True)

        @pl.when(i == NBLK - 1)
        def _():
            pooled = sums_ref[...] / jnp.maximum(cnts_ref[...], 1.0)
            o_ref[...] = jnp.dot(pooled, wl_ref[...],
                                 preferred_element_type=jnp.float32) + bl_ref[...]

    return pl.pallas_call(
        body,
        grid=(NBLK,),
        in_specs=[pl.BlockSpec((NC, NB, HH), lambda i: (0, i, 0)),
                  pl.BlockSpec((NC, NB, HH), lambda i: (0, i, 0)),
                  pl.BlockSpec((NB, 1), lambda i: (i, 0)),
                  pl.BlockSpec((1, H), lambda i: (0, 0)),
                  pl.BlockSpec((1, 1, NB), lambda i: (i, 0, 0)),
                  pl.BlockSpec((H, C), lambda i: (0, 0)),
                  pl.BlockSpec((1, C), lambda i: (0, 0))],
        out_specs=pl.BlockSpec((G, C), lambda i: (0, 0)),
        out_shape=jax.ShapeDtypeStruct((G, C), jnp.float32),
        scratch_shapes=[pltpu.VMEM((G, H), jnp.float32),
                        pltpu.VMEM((G, 1), jnp.float32)],
    )(s, y, dinv, b, batch3, Wl, bl)


def _pad_rows(y):
    return jnp.concatenate(
        [y, jnp.zeros((NC, NPAD - N, HH), jnp.float32)], axis=1)


def kernel(x, edge_index, batch, W1, b1, W2, b2, W3, b3, Wl, bl):
    pad = EPAD - E
    srcp = jnp.concatenate([edge_index[0],
                            jnp.full((pad,), N, jnp.int32)]).reshape(TCH, K)
    dstp = jnp.concatenate([edge_index[1],
                            jnp.full((pad,), N, jnp.int32)]).reshape(TCH, K)
    zeros_h = jnp.zeros((RPS, HH), jnp.float32)
    zeros16 = jnp.zeros((RPS, 16), jnp.float32)
    ones16 = jnp.ones((K, 16), jnp.float32)
    batch3 = batch.reshape(NBLK, 1, NB)

    parts16 = _sc_deg(dstp, ones16, zeros16)
    xw1 = _tc_mm1(x, W1)
    dinv, y1 = _tc_prep(parts16[:, :N], xw1)

    s1 = _sc_agg(_pad_rows(y1), srcp, dstp, zeros_h)
    y2 = _tc_combine(s1[:, :N], y1, dinv, b1.reshape(1, H), W2)
    s2 = _sc_agg(_pad_rows(y2), srcp, dstp, zeros_h)
    y3 = _tc_combine(s2[:, :N], y2, dinv, b2.reshape(1, H), W3)
    s3 = _sc_agg(_pad_rows(y3), srcp, dstp, zeros_h)
    return _tc_pool(s3[:, :N], y3, dinv, b3.reshape(1, H), batch3,
                    Wl, bl.reshape(1, C))
